# Initial kernel scaffold; baseline (speedup 1.0000x reference)
import jax
import jax.numpy as jnp
from jax.experimental import pallas as pl


def kernel(x, edge_index, batch, W1, b1, W2, b2, W3, b3, Wh, bh):
    raise NotImplementedError("write your pallas kernel here")


# speedup vs baseline: 6.6167x; 6.6167x over previous
import functools

import jax
import jax.numpy as jnp
from jax import lax
from jax.experimental import pallas as pl
from jax.experimental.pallas import tpu as pltpu
from jax.experimental.pallas import tpu_sc as plsc

N = 10000
E = 160000
D = 256
DH = 128
NG = 128
BLK = 128
EBLK_TOTAL = 1280
E_PAD = EBLK_TOTAL * BLK
N_PAD = 10112
ROWS_PER_SUB = N_PAD // 16
DEG_BLOCKS = EBLK_TOTAL // 32
AGG_BLOCKS = EBLK_TOTAL // 16

@functools.cache
def _sc_kernels():
    mesh = plsc.VectorSubcoreMesh(core_axis_name="c", subcore_axis_name="s",
                                  num_cores=2)

    @functools.partial(
        pl.kernel,
        mesh=mesh,
        out_type=jax.ShapeDtypeStruct((2, N_PAD, DH), jnp.float32),
        scratch_types=[
            pltpu.VMEM((DEG_BLOCKS, BLK), jnp.int32),
            pltpu.VMEM((BLK, DH), jnp.float32),
            pltpu.VMEM_SHARED((N_PAD, DH), jnp.float32),
        ],
    )
    def _deg_kernel(edges_hbm, ones_hbm, zeros_hbm, out_hbm, dst_idx, ones_v, deg_sh):
        c = lax.axis_index("c")
        s = lax.axis_index("s")
        wid = s * 2 + c
        pltpu.sync_copy(ones_hbm, ones_v)
        pltpu.sync_copy(zeros_hbm.at[pl.ds(s * ROWS_PER_SUB, ROWS_PER_SUB)],
                        deg_sh.at[pl.ds(s * ROWS_PER_SUB, ROWS_PER_SUB)])
        pltpu.sync_copy(edges_hbm.at[1, pl.ds(wid * DEG_BLOCKS, DEG_BLOCKS)], dst_idx)
        plsc.subcore_barrier()

        def body(j, carry):
            pltpu.sync_copy(ones_v, deg_sh.at[dst_idx.at[j]], add=True)
            return carry

        lax.fori_loop(0, DEG_BLOCKS, body, 0)
        plsc.subcore_barrier()
        pltpu.sync_copy(deg_sh.at[pl.ds(s * ROWS_PER_SUB, ROWS_PER_SUB)],
                        out_hbm.at[c, pl.ds(s * ROWS_PER_SUB, ROWS_PER_SUB)])

    @functools.partial(
        pl.kernel,
        mesh=mesh,
        out_type=jax.ShapeDtypeStruct((2, N_PAD, DH), jnp.float32),
        scratch_types=[
            pltpu.VMEM((AGG_BLOCKS, BLK), jnp.int32),
            pltpu.VMEM((AGG_BLOCKS, BLK), jnp.int32),
            pltpu.VMEM((BLK, DH), jnp.float32),
            pltpu.VMEM_SHARED((N_PAD, DH), jnp.float32),
            pltpu.SemaphoreType.DMA,
        ],
    )
    def _agg_kernel(edges_hbm, hlo_hbm, hhi_hbm, zeros_hbm, out_hbm,
                    src_idx, dst_idx, rows, agg_sh, sem):
        c = lax.axis_index("c")
        s = lax.axis_index("s")
        blk0 = s * AGG_BLOCKS
        pltpu.sync_copy(edges_hbm.at[0, pl.ds(blk0, AGG_BLOCKS)], src_idx)
        pltpu.sync_copy(edges_hbm.at[1, pl.ds(blk0, AGG_BLOCKS)], dst_idx)
        pltpu.sync_copy(zeros_hbm.at[pl.ds(s * ROWS_PER_SUB, ROWS_PER_SUB)],
                        agg_sh.at[pl.ds(s * ROWS_PER_SUB, ROWS_PER_SUB)])
        plsc.subcore_barrier()

        def run(table_hbm):
            def body(j, carry):
                pltpu.async_copy(table_hbm.at[src_idx.at[j]], rows, sem).wait()
                pltpu.sync_copy(rows, agg_sh.at[dst_idx.at[j]], add=True)
                return carry
            lax.fori_loop(0, AGG_BLOCKS, body, 0)

        @pl.when(c == 0)
        def _():
            run(hlo_hbm)

        @pl.when(c == 1)
        def _():
            run(hhi_hbm)

        plsc.subcore_barrier()
        pltpu.sync_copy(agg_sh.at[pl.ds(s * ROWS_PER_SUB, ROWS_PER_SUB)],
                        out_hbm.at[c, pl.ds(s * ROWS_PER_SUB, ROWS_PER_SUB)])

    return _deg_kernel, _agg_kernel


_RB = 2000


def _tc_first_body(deg16_ref, x_ref, w_ref, dinv_ref, hlo_ref, hhi_ref):
    deg = deg16_ref[0, :, 0:1] + deg16_ref[1, :, 0:1] + 1.0
    dinv = lax.rsqrt(deg)
    dinv_ref[...] = dinv
    ht = dinv * jnp.dot(x_ref[...], w_ref[...], preferred_element_type=jnp.float32)
    hlo_ref[...] = ht[:, :DH]
    hhi_ref[...] = ht[:, DH:]


def _tc_first(deg16, x, w):
    return pl.pallas_call(
        _tc_first_body,
        grid=(N // _RB,),
        in_specs=[
            pl.BlockSpec((2, _RB, DH), lambda i: (0, i, 0)),
            pl.BlockSpec((_RB, D), lambda i: (i, 0)),
            pl.BlockSpec((D, D), lambda i: (0, 0)),
        ],
        out_specs=[
            pl.BlockSpec((_RB, 1), lambda i: (i, 0)),
            pl.BlockSpec((_RB, DH), lambda i: (i, 0)),
            pl.BlockSpec((_RB, DH), lambda i: (i, 0)),
        ],
        out_shape=[
            jax.ShapeDtypeStruct((N, 1), jnp.float32),
            jax.ShapeDtypeStruct((N, DH), jnp.float32),
            jax.ShapeDtypeStruct((N, DH), jnp.float32),
        ],
    )(deg16, x, w)


def _tc_mid_body(agg_ref, hlo_ref, hhi_ref, dinv_ref, b_ref, w_ref,
                 hlo2_ref, hhi2_ref):
    dinv = dinv_ref[...]
    h = jnp.concatenate([agg_ref[0] + hlo_ref[...], agg_ref[1] + hhi_ref[...]],
                        axis=1)
    h = jnp.maximum(dinv * h + b_ref[...], 0.0)
    ht = dinv * jnp.dot(h, w_ref[...], preferred_element_type=jnp.float32)
    hlo2_ref[...] = ht[:, :DH]
    hhi2_ref[...] = ht[:, DH:]


def _tc_mid(agg, hlo, hhi, dinv, b, w):
    return pl.pallas_call(
        _tc_mid_body,
        grid=(N // _RB,),
        in_specs=[
            pl.BlockSpec((2, _RB, DH), lambda i: (0, i, 0)),
            pl.BlockSpec((_RB, DH), lambda i: (i, 0)),
            pl.BlockSpec((_RB, DH), lambda i: (i, 0)),
            pl.BlockSpec((_RB, 1), lambda i: (i, 0)),
            pl.BlockSpec((1, D), lambda i: (0, 0)),
            pl.BlockSpec((D, D), lambda i: (0, 0)),
        ],
        out_specs=[
            pl.BlockSpec((_RB, DH), lambda i: (i, 0)),
            pl.BlockSpec((_RB, DH), lambda i: (i, 0)),
        ],
        out_shape=[
            jax.ShapeDtypeStruct((N, DH), jnp.float32),
            jax.ShapeDtypeStruct((N, DH), jnp.float32),
        ],
    )(agg, hlo, hhi, dinv, b, w)


def _tc_final_body(agg_ref, hlo_ref, hhi_ref, dinv_ref, b_ref, batch_ref,
                   wh_ref, bh_ref, out_ref, ys_acc, cnt_acc):
    i = pl.program_id(0)
    dinv = dinv_ref[...]
    h = jnp.concatenate([agg_ref[0] + hlo_ref[...], agg_ref[1] + hhi_ref[...]],
                        axis=1)
    h = jnp.maximum(dinv * h + b_ref[...], 0.0)
    y = jnp.dot(h, wh_ref[...], preferred_element_type=jnp.float32)
    g = lax.broadcasted_iota(jnp.int32, (NG, _RB), 0)
    m = (batch_ref[0] == g).astype(jnp.float32)
    ys = jnp.dot(m, y, preferred_element_type=jnp.float32)
    cnt = jnp.sum(m, axis=1, keepdims=True)

    @pl.when(i == 0)
    def _():
        ys_acc[...] = jnp.zeros_like(ys_acc)
        cnt_acc[...] = jnp.zeros_like(cnt_acc)

    ys_acc[...] += ys
    cnt_acc[...] += cnt

    @pl.when(i == (N // _RB) - 1)
    def _():
        out_ref[...] = ys_acc[...] / jnp.maximum(cnt_acc[...], 1.0) + bh_ref[...]


def _tc_final(agg, hlo, hhi, dinv, b, batch_r, wh, bh):
    return pl.pallas_call(
        _tc_final_body,
        grid=(N // _RB,),
        in_specs=[
            pl.BlockSpec((2, _RB, DH), lambda i: (0, i, 0)),
            pl.BlockSpec((_RB, DH), lambda i: (i, 0)),
            pl.BlockSpec((_RB, DH), lambda i: (i, 0)),
            pl.BlockSpec((_RB, 1), lambda i: (i, 0)),
            pl.BlockSpec((1, D), lambda i: (0, 0)),
            pl.BlockSpec((1, 1, _RB), lambda i: (i, 0, 0)),
            pl.BlockSpec((D, 1), lambda i: (0, 0)),
            pl.BlockSpec((1, 1), lambda i: (0, 0)),
        ],
        out_specs=pl.BlockSpec((NG, 1), lambda i: (0, 0)),
        out_shape=jax.ShapeDtypeStruct((NG, 1), jnp.float32),
        scratch_shapes=[
            pltpu.VMEM((NG, 1), jnp.float32),
            pltpu.VMEM((NG, 1), jnp.float32),
        ],
    )(agg, hlo, hhi, dinv, b, batch_r, wh, bh)


def kernel(x, edge_index, batch, W1, b1, W2, b2, W3, b3, Wh, bh):
    e = edge_index.astype(jnp.int32)
    pad = jnp.concatenate(
        [jnp.zeros((1, E_PAD - E), jnp.int32),
         jnp.full((1, E_PAD - E), N, jnp.int32)], axis=0)
    e = jnp.concatenate([e, pad], axis=1).reshape(2, EBLK_TOTAL, BLK)
    batch_r = batch.astype(jnp.int32).reshape(N // _RB, 1, _RB)
    zeros_h = jnp.zeros((N_PAD, DH), jnp.float32)
    ones_w = jnp.ones((BLK, DH), jnp.float32)

    deg_kernel, agg_kernel = _sc_kernels()
    deg16 = deg_kernel(e, ones_w, zeros_h)
    dinv, hlo, hhi = _tc_first(deg16, x, W1)
    agg = agg_kernel(e, hlo, hhi, zeros_h)
    hlo, hhi = _tc_mid(agg, hlo, hhi, dinv, b1.reshape(1, D), W2)
    agg = agg_kernel(e, hlo, hhi, zeros_h)
    hlo, hhi = _tc_mid(agg, hlo, hhi, dinv, b2.reshape(1, D), W3)
    agg = agg_kernel(e, hlo, hhi, zeros_h)
    return _tc_final(agg, hlo, hhi, dinv, b3.reshape(1, D), batch_r,
                     Wh.reshape(D, 1), bh.reshape(1, 1))


# speedup vs baseline: 7.4794x; 1.1304x over previous
import functools

import jax
import jax.numpy as jnp
from jax import lax
from jax.experimental import pallas as pl
from jax.experimental.pallas import tpu as pltpu
from jax.experimental.pallas import tpu_sc as plsc

N = 10000
E = 160000
D = 256
DH = 128
NG = 128
BLK = 128
EBLK_TOTAL = 1280
E_PAD = EBLK_TOTAL * BLK
N_PAD = 10112
ROWS_PER_SUB = N_PAD // 16
DEG_BLOCKS = EBLK_TOTAL // 32
AGG_BLOCKS = EBLK_TOTAL // 16
CHUNK = 8

@functools.cache
def _sc_kernels():
    mesh = plsc.VectorSubcoreMesh(core_axis_name="c", subcore_axis_name="s",
                                  num_cores=2)

    @functools.partial(
        pl.kernel,
        mesh=mesh,
        out_type=jax.ShapeDtypeStruct((2, N_PAD, DH), jnp.float32),
        scratch_types=[
            pltpu.VMEM((DEG_BLOCKS, BLK), jnp.int32),
            pltpu.VMEM((BLK, DH), jnp.float32),
            pltpu.VMEM_SHARED((N_PAD, DH), jnp.float32),
        ],
    )
    def _deg_kernel(edges_hbm, ones_hbm, zeros_hbm, out_hbm, dst_idx, ones_v, deg_sh):
        c = lax.axis_index("c")
        s = lax.axis_index("s")
        wid = s * 2 + c
        pltpu.sync_copy(ones_hbm, ones_v)
        pltpu.sync_copy(zeros_hbm.at[pl.ds(s * ROWS_PER_SUB, ROWS_PER_SUB)],
                        deg_sh.at[pl.ds(s * ROWS_PER_SUB, ROWS_PER_SUB)])
        pltpu.sync_copy(edges_hbm.at[1, pl.ds(wid * DEG_BLOCKS, DEG_BLOCKS)], dst_idx)
        plsc.subcore_barrier()

        def body(j, carry):
            pltpu.sync_copy(ones_v, deg_sh.at[dst_idx.at[j]], add=True)
            return carry

        lax.fori_loop(0, DEG_BLOCKS, body, 0)
        plsc.subcore_barrier()
        pltpu.sync_copy(deg_sh.at[pl.ds(s * ROWS_PER_SUB, ROWS_PER_SUB)],
                        out_hbm.at[c, pl.ds(s * ROWS_PER_SUB, ROWS_PER_SUB)])

    @functools.partial(
        pl.kernel,
        mesh=mesh,
        out_type=jax.ShapeDtypeStruct((2, N_PAD, DH), jnp.float32),
        scratch_types=[
            pltpu.VMEM((2, CHUNK, BLK), jnp.int32),
            pltpu.VMEM((BLK, DH), jnp.float32),
            pltpu.VMEM((BLK, DH), jnp.float32),
            pltpu.SemaphoreType.DMA,
            pltpu.SemaphoreType.DMA,
            pltpu.SemaphoreType.DMA,
            pltpu.VMEM_SHARED((N_PAD, DH), jnp.float32),
        ],
    )
    def _agg_kernel(edges_hbm, hlo_hbm, hhi_hbm, zeros_hbm, out_hbm,
                    idx, r0, r1, si, sg0, sg1, agg_sh):
        c = lax.axis_index("c")
        s = lax.axis_index("s")
        blk0 = s * AGG_BLOCKS
        nchunks = AGG_BLOCKS // CHUNK

        pltpu.async_copy(edges_hbm.at[:, pl.ds(blk0, CHUNK)], idx, si)
        pltpu.sync_copy(zeros_hbm.at[pl.ds(s * ROWS_PER_SUB, ROWS_PER_SUB)],
                        agg_sh.at[pl.ds(s * ROWS_PER_SUB, ROWS_PER_SUB)])
        plsc.subcore_barrier()

        rows = (r0, r1)
        sems = (sg0, sg1)

        def run(table_hbm):
            def start(b):
                pltpu.async_copy(table_hbm.at[idx.at[0, b]], rows[b % 2],
                                 sems[b % 2])

            def wait(b):
                pltpu.make_async_copy(table_hbm.at[idx.at[0, 0]], rows[b % 2],
                                      sems[b % 2]).wait()

            def body(g, carry):
                pltpu.make_async_copy(edges_hbm.at[:, pl.ds(blk0, CHUNK)],
                                      idx, si).wait()
                start(0)
                start(1)
                for b in range(CHUNK):
                    wait(b)
                    pltpu.sync_copy(rows[b % 2], agg_sh.at[idx.at[1, b]],
                                    add=True)
                    if b + 2 < CHUNK:
                        start(b + 2)
                nxt = jnp.minimum(g + 1, nchunks - 1)
                pltpu.async_copy(
                    edges_hbm.at[:, pl.ds(blk0 + nxt * CHUNK, CHUNK)], idx, si)
                return carry

            lax.fori_loop(0, nchunks, body, 0)
            pltpu.make_async_copy(edges_hbm.at[:, pl.ds(blk0, CHUNK)],
                                  idx, si).wait()

        @pl.when(c == 0)
        def _():
            run(hlo_hbm)

        @pl.when(c == 1)
        def _():
            run(hhi_hbm)

        plsc.subcore_barrier()
        pltpu.sync_copy(agg_sh.at[pl.ds(s * ROWS_PER_SUB, ROWS_PER_SUB)],
                        out_hbm.at[c, pl.ds(s * ROWS_PER_SUB, ROWS_PER_SUB)])

    return _deg_kernel, _agg_kernel


_RB = 2000


def _tc_first_body(deg16_ref, x_ref, w_ref, dinv_ref, hlo_ref, hhi_ref):
    deg = deg16_ref[0, :, 0:1] + deg16_ref[1, :, 0:1] + 1.0
    dinv = lax.rsqrt(deg)
    dinv_ref[...] = dinv
    ht = dinv * jnp.dot(x_ref[...], w_ref[...], preferred_element_type=jnp.float32)
    hlo_ref[...] = ht[:, :DH]
    hhi_ref[...] = ht[:, DH:]


def _tc_first(deg16, x, w):
    return pl.pallas_call(
        _tc_first_body,
        grid=(N // _RB,),
        in_specs=[
            pl.BlockSpec((2, _RB, DH), lambda i: (0, i, 0)),
            pl.BlockSpec((_RB, D), lambda i: (i, 0)),
            pl.BlockSpec((D, D), lambda i: (0, 0)),
        ],
        out_specs=[
            pl.BlockSpec((_RB, 1), lambda i: (i, 0)),
            pl.BlockSpec((_RB, DH), lambda i: (i, 0)),
            pl.BlockSpec((_RB, DH), lambda i: (i, 0)),
        ],
        out_shape=[
            jax.ShapeDtypeStruct((N, 1), jnp.float32),
            jax.ShapeDtypeStruct((N, DH), jnp.float32),
            jax.ShapeDtypeStruct((N, DH), jnp.float32),
        ],
    )(deg16, x, w)


def _tc_mid_body(agg_ref, hlo_ref, hhi_ref, dinv_ref, b_ref, w_ref,
                 hlo2_ref, hhi2_ref):
    dinv = dinv_ref[...]
    h = jnp.concatenate([agg_ref[0] + hlo_ref[...], agg_ref[1] + hhi_ref[...]],
                        axis=1)
    h = jnp.maximum(dinv * h + b_ref[...], 0.0)
    ht = dinv * jnp.dot(h, w_ref[...], preferred_element_type=jnp.float32)
    hlo2_ref[...] = ht[:, :DH]
    hhi2_ref[...] = ht[:, DH:]


def _tc_mid(agg, hlo, hhi, dinv, b, w):
    return pl.pallas_call(
        _tc_mid_body,
        grid=(N // _RB,),
        in_specs=[
            pl.BlockSpec((2, _RB, DH), lambda i: (0, i, 0)),
            pl.BlockSpec((_RB, DH), lambda i: (i, 0)),
            pl.BlockSpec((_RB, DH), lambda i: (i, 0)),
            pl.BlockSpec((_RB, 1), lambda i: (i, 0)),
            pl.BlockSpec((1, D), lambda i: (0, 0)),
            pl.BlockSpec((D, D), lambda i: (0, 0)),
        ],
        out_specs=[
            pl.BlockSpec((_RB, DH), lambda i: (i, 0)),
            pl.BlockSpec((_RB, DH), lambda i: (i, 0)),
        ],
        out_shape=[
            jax.ShapeDtypeStruct((N, DH), jnp.float32),
            jax.ShapeDtypeStruct((N, DH), jnp.float32),
        ],
    )(agg, hlo, hhi, dinv, b, w)


def _tc_final_body(agg_ref, hlo_ref, hhi_ref, dinv_ref, b_ref, batch_ref,
                   wh_ref, bh_ref, out_ref, ys_acc, cnt_acc):
    i = pl.program_id(0)
    dinv = dinv_ref[...]
    h = jnp.concatenate([agg_ref[0] + hlo_ref[...], agg_ref[1] + hhi_ref[...]],
                        axis=1)
    h = jnp.maximum(dinv * h + b_ref[...], 0.0)
    y = jnp.dot(h, wh_ref[...], preferred_element_type=jnp.float32)
    g = lax.broadcasted_iota(jnp.int32, (NG, _RB), 0)
    m = (batch_ref[0] == g).astype(jnp.float32)
    ys = jnp.dot(m, y, preferred_element_type=jnp.float32)
    cnt = jnp.sum(m, axis=1, keepdims=True)

    @pl.when(i == 0)
    def _():
        ys_acc[...] = jnp.zeros_like(ys_acc)
        cnt_acc[...] = jnp.zeros_like(cnt_acc)

    ys_acc[...] += ys
    cnt_acc[...] += cnt

    @pl.when(i == (N // _RB) - 1)
    def _():
        out_ref[...] = ys_acc[...] / jnp.maximum(cnt_acc[...], 1.0) + bh_ref[...]


def _tc_final(agg, hlo, hhi, dinv, b, batch_r, wh, bh):
    return pl.pallas_call(
        _tc_final_body,
        grid=(N // _RB,),
        in_specs=[
            pl.BlockSpec((2, _RB, DH), lambda i: (0, i, 0)),
            pl.BlockSpec((_RB, DH), lambda i: (i, 0)),
            pl.BlockSpec((_RB, DH), lambda i: (i, 0)),
            pl.BlockSpec((_RB, 1), lambda i: (i, 0)),
            pl.BlockSpec((1, D), lambda i: (0, 0)),
            pl.BlockSpec((1, 1, _RB), lambda i: (i, 0, 0)),
            pl.BlockSpec((D, 1), lambda i: (0, 0)),
            pl.BlockSpec((1, 1), lambda i: (0, 0)),
        ],
        out_specs=pl.BlockSpec((NG, 1), lambda i: (0, 0)),
        out_shape=jax.ShapeDtypeStruct((NG, 1), jnp.float32),
        scratch_shapes=[
            pltpu.VMEM((NG, 1), jnp.float32),
            pltpu.VMEM((NG, 1), jnp.float32),
        ],
    )(agg, hlo, hhi, dinv, b, batch_r, wh, bh)


def kernel(x, edge_index, batch, W1, b1, W2, b2, W3, b3, Wh, bh):
    e = edge_index.astype(jnp.int32)
    pad = jnp.concatenate(
        [jnp.zeros((1, E_PAD - E), jnp.int32),
         jnp.full((1, E_PAD - E), N, jnp.int32)], axis=0)
    e = jnp.concatenate([e, pad], axis=1).reshape(2, EBLK_TOTAL, BLK)
    batch_r = batch.astype(jnp.int32).reshape(N // _RB, 1, _RB)
    zeros_h = jnp.zeros((N_PAD, DH), jnp.float32)
    ones_w = jnp.ones((BLK, DH), jnp.float32)

    deg_kernel, agg_kernel = _sc_kernels()
    deg16 = deg_kernel(e, ones_w, zeros_h)
    dinv, hlo, hhi = _tc_first(deg16, x, W1)
    agg = agg_kernel(e, hlo, hhi, zeros_h)
    hlo, hhi = _tc_mid(agg, hlo, hhi, dinv, b1.reshape(1, D), W2)
    agg = agg_kernel(e, hlo, hhi, zeros_h)
    hlo, hhi = _tc_mid(agg, hlo, hhi, dinv, b2.reshape(1, D), W3)
    agg = agg_kernel(e, hlo, hhi, zeros_h)
    return _tc_final(agg, hlo, hhi, dinv, b3.reshape(1, D), batch_r,
                     Wh.reshape(D, 1), bh.reshape(1, 1))
